# baseline (speedup 1.0000x reference)
import jax
import jax.numpy as jnp
import numpy as np
from jax.experimental import pallas as pl
from jax.experimental.pallas import tpu as pltpu

_LOG2 = float(np.log(2.0))
_LOG2E = float(np.log2(np.e))


def _mlp_pool_kernel(x_ref, mask_ref, w1_ref, b1_ref, w2_ref, c2_ref, out_ref):
    bb, a, d = x_ref.shape
    n = bb * a
    x = x_ref[...].reshape(n, d)
    t = jnp.dot(x, w1_ref[...], preferred_element_type=jnp.float32) + b1_ref[...]
    u = (jnp.maximum(t, 0.0) - 1.0) + jnp.log2(1.0 + jnp.exp2(-jnp.abs(t)))
    mask = mask_ref[...]
    mask_tiled = jnp.concatenate([mask] * bb, axis=1)
    seg = jax.lax.broadcasted_iota(jnp.int32, (bb, n), 1) // a
    row = jax.lax.broadcasted_iota(jnp.int32, (bb, n), 0)
    mt = jnp.where(seg == row, mask_tiled, 0.0)
    q = jnp.dot(mt, u, preferred_element_type=jnp.float32)
    y = jnp.sum(q * w2_ref[...], axis=1, keepdims=True)
    msum = jnp.sum(mask, axis=1, keepdims=True)
    out_ref[...] = y + c2_ref[0, 0] * msum


def kernel(representation, atomic_numbers, atom_mask, W1, b1, W2, b2):
    B, A, D = representation.shape
    H = W1.shape[1]
    BB = 32

    w1s = W1 * _LOG2E
    b1s = (b1 * _LOG2E).reshape(1, H)
    w2l = (W2 * _LOG2).reshape(1, H)
    c2 = b2.reshape(1, 1)

    y = pl.pallas_call(
        _mlp_pool_kernel,
        grid=(B // BB,),
        in_specs=[
            pl.BlockSpec((BB, A, D), lambda i: (i, 0, 0)),
            pl.BlockSpec((BB, A), lambda i: (i, 0)),
            pl.BlockSpec((D, H), lambda i: (0, 0)),
            pl.BlockSpec((1, H), lambda i: (0, 0)),
            pl.BlockSpec((1, H), lambda i: (0, 0)),
            pl.BlockSpec((1, 1), lambda i: (0, 0)),
        ],
        out_specs=pl.BlockSpec((BB, 1), lambda i: (i, 0)),
        out_shape=jax.ShapeDtypeStruct((B, 1), jnp.float32),
        compiler_params=pltpu.CompilerParams(
            vmem_limit_bytes=120 * 1024 * 1024,
        ),
    )(representation, atom_mask, w1s, b1s, w2l, c2)
    return y
